# baseline (speedup 1.0000x reference)
import functools

import jax
import jax.numpy as jnp
from jax import lax
from jax.experimental import pallas as pl
from jax.experimental.pallas import tpu as pltpu
from jax.experimental.pallas import tpu_sc as plsc

_NUM_CORES = 2
_NUM_SUBCORES = 16
_CHUNK_ROWS = 16
_NBUF = 4


@functools.lru_cache(maxsize=None)
def _make_copy_kernel(seq_len: int, dim: int):
    nworkers = _NUM_CORES * _NUM_SUBCORES
    rows_per_w = seq_len // nworkers
    chunk = min(rows_per_w, _CHUNK_ROWS)
    nchunk = rows_per_w // chunk
    nbuf = min(_NBUF, nchunk)
    mesh = plsc.VectorSubcoreMesh(core_axis_name="c", subcore_axis_name="s")

    @functools.partial(
        pl.kernel,
        mesh=mesh,
        out_type=jax.ShapeDtypeStruct((seq_len, dim), jnp.float32),
        scratch_types=[
            pltpu.VMEM((nbuf, chunk, dim), jnp.float32),
        ]
        + [pltpu.SemaphoreType.DMA] * (2 * nbuf),
    )
    def k(emb_hbm, out_hbm, buf, *sems):
        rsems = sems[:nbuf]
        wsems = sems[nbuf:]
        wid = lax.axis_index("s") * _NUM_CORES + lax.axis_index("c")
        base = wid * rows_per_w

        def read(c):
            b = c % nbuf
            return pltpu.async_copy(
                emb_hbm.at[pl.ds(base + c * chunk, chunk)],
                buf.at[b], rsems[b])

        def write(c):
            b = c % nbuf
            return pltpu.async_copy(
                buf.at[b],
                out_hbm.at[pl.ds(base + c * chunk, chunk)], wsems[b])

        reads = {}
        writes = {}
        for c in range(min(nbuf - 1, nchunk)):
            reads[c] = read(c)
        for c in range(nchunk):
            if c + nbuf - 1 < nchunk:
                if c - 1 >= 0:
                    writes.pop(c - 1).wait()
                reads[c + nbuf - 1] = read(c + nbuf - 1)
            reads.pop(c).wait()
            writes[c] = write(c)
        for w in writes.values():
            w.wait()

    return k


def kernel(x, emb_weight):
    seq_len = x.shape[1]
    dim = emb_weight.shape[1]
    return _make_copy_kernel(seq_len, dim)(emb_weight)


# speedup vs baseline: 1.0259x; 1.0259x over previous
import functools

import jax
import jax.numpy as jnp
from jax import lax
from jax.experimental import pallas as pl
from jax.experimental.pallas import tpu as pltpu
from jax.experimental.pallas import tpu_sc as plsc

_NUM_CORES = 2
_NUM_SUBCORES = 16
_CHUNK_ROWS = 32
_NBUF = 3


@functools.lru_cache(maxsize=None)
def _make_copy_kernel(seq_len: int, dim: int):
    nworkers = _NUM_CORES * _NUM_SUBCORES
    rows_per_w = seq_len // nworkers
    chunk = min(rows_per_w, _CHUNK_ROWS)
    nchunk = rows_per_w // chunk
    nbuf = min(_NBUF, nchunk)
    mesh = plsc.VectorSubcoreMesh(core_axis_name="c", subcore_axis_name="s")

    @functools.partial(
        pl.kernel,
        mesh=mesh,
        out_type=jax.ShapeDtypeStruct((seq_len, dim), jnp.float32),
        scratch_types=[
            pltpu.VMEM((nbuf, chunk, dim), jnp.float32),
        ]
        + [pltpu.SemaphoreType.DMA] * (2 * nbuf),
    )
    def k(emb_hbm, out_hbm, buf, *sems):
        rsems = sems[:nbuf]
        wsems = sems[nbuf:]
        wid = lax.axis_index("s") * _NUM_CORES + lax.axis_index("c")
        base = wid * rows_per_w

        def read(c):
            b = c % nbuf
            return pltpu.async_copy(
                emb_hbm.at[pl.ds(base + c * chunk, chunk)],
                buf.at[b], rsems[b])

        def write(c):
            b = c % nbuf
            return pltpu.async_copy(
                buf.at[b],
                out_hbm.at[pl.ds(base + c * chunk, chunk)], wsems[b])

        reads = {}
        writes = {}
        for c in range(min(nbuf - 1, nchunk)):
            reads[c] = read(c)
        for c in range(nchunk):
            if c + nbuf - 1 < nchunk:
                if c - 1 >= 0:
                    writes.pop(c - 1).wait()
                reads[c + nbuf - 1] = read(c + nbuf - 1)
            reads.pop(c).wait()
            writes[c] = write(c)
        for w in writes.values():
            w.wait()

    return k


def kernel(x, emb_weight):
    seq_len = x.shape[1]
    dim = emb_weight.shape[1]
    return _make_copy_kernel(seq_len, dim)(emb_weight)
